# baseline (speedup 1.0000x reference)
import functools

import jax
import jax.numpy as jnp
from jax import lax
from jax.experimental import pallas as pl
from jax.experimental.pallas import tpu as pltpu
from jax.experimental.pallas import tpu_sc as plsc

BN_EPS = 1e-3

NC = 2
NS = 16
NW = NC * NS
CHUNK = 128


def _sc_gather(table, idx3, n_rows, embed):
    n_chunks = idx3.shape[1]
    b_per_w = n_chunks * CHUNK
    mesh = plsc.VectorSubcoreMesh(core_axis_name="c", subcore_axis_name="s")

    @functools.partial(
        pl.kernel,
        mesh=mesh,
        compiler_params=pltpu.CompilerParams(use_tc_tiling_on_sc=False),
        out_type=jax.ShapeDtypeStruct((n_rows, embed), jnp.float32),
        scratch_types=[
            pltpu.VMEM((n_chunks, CHUNK), jnp.int32),
            pltpu.VMEM((b_per_w, embed), jnp.float32),
            pltpu.SemaphoreType.DMA,
        ],
    )
    def gather_kernel(table_hbm, idx_hbm, out_hbm, idx_v, rows_v, sem):
        wid = lax.axis_index("s") * NC + lax.axis_index("c")
        pltpu.sync_copy(idx_hbm.at[wid], idx_v)
        copies = [
            pltpu.async_copy(
                table_hbm.at[idx_v.at[j]],
                rows_v.at[pl.ds(j * CHUNK, CHUNK)],
                sem,
            )
            for j in range(n_chunks)
        ]
        for c in copies:
            c.wait()
        pltpu.sync_copy(rows_v, out_hbm.at[pl.ds(wid * b_per_w, b_per_w)])

    return gather_kernel(table, idx3)


def _mlp_body(x_ref, w1_ref, b1_ref, g_ref, be_ref, mm_ref, mv_ref,
              w2_ref, b2_ref, o_ref):
    h = jnp.dot(x_ref[...], w1_ref[...], preferred_element_type=jnp.float32)
    h = jnp.maximum(h + b1_ref[...], 0.0)
    s = g_ref[...] * lax.rsqrt(mv_ref[...] + BN_EPS)
    t = be_ref[...] - s * mm_ref[...]
    h = h * s + t
    o_ref[...] = (
        jnp.dot(h, w2_ref[...], preferred_element_type=jnp.float32)
        + b2_ref[...]
    )


def _tc_mlp(x, W1, b1, gamma, beta, mmean, mvar, W2, b2, block_m):
    n, embed = x.shape
    hdim = W2.shape[1]
    row = lambda v: v.reshape(1, -1)
    vec_spec = pl.BlockSpec((1, hdim), lambda i: (0, 0))
    return pl.pallas_call(
        _mlp_body,
        grid=(n // block_m,),
        in_specs=[
            pl.BlockSpec((block_m, embed), lambda i: (i, 0)),
            pl.BlockSpec((embed, hdim), lambda i: (0, 0)),
            vec_spec, vec_spec, vec_spec, vec_spec, vec_spec,
            pl.BlockSpec((hdim, hdim), lambda i: (0, 0)),
            vec_spec,
        ],
        out_specs=pl.BlockSpec((block_m, hdim), lambda i: (i, 0)),
        out_shape=jax.ShapeDtypeStruct((n, hdim), jnp.float32),
    )(x, W1, row(b1), row(gamma), row(beta), row(mmean), row(mvar),
      W2, row(b2))


def kernel(anchor, positive, negative, emb_table, W1, b1, gamma, beta,
           moving_mean, moving_var, W2, b2):
    b = anchor.shape[0]
    nb = 3 * b
    idx = jnp.concatenate([anchor, positive, negative]).astype(jnp.int32)
    idx3 = idx.reshape(NW, nb // (NW * CHUNK), CHUNK)
    gathered = _sc_gather(emb_table, idx3, nb, emb_table.shape[1])
    out = _tc_mlp(gathered, W1, b1, gamma, beta, moving_mean, moving_var,
                  W2, b2, block_m=2048)
    return (out[:b], out[b:2 * b], out[2 * b:])
